# baseline (speedup 1.0000x reference)
import jax
import jax.numpy as jnp
from jax.experimental import pallas as pl


def _matmul_block(x_ref, m_ref, o_ref):
    o_ref[...] = jax.lax.dot_general(
        x_ref[...],
        m_ref[...],
        dimension_numbers=(((1,), (1,)), ((), ())),
        preferred_element_type=jnp.float32,
    )


def kernel(x, M):
    B, C, N = x.shape
    O = M.shape[0]
    BC = B * C
    x2 = x.reshape(BC, N)

    OT = 512
    y = pl.pallas_call(
        _matmul_block,
        grid=(pl.cdiv(O, OT),),
        in_specs=[
            pl.BlockSpec((BC, N), lambda i: (0, 0)),
            pl.BlockSpec((OT, N), lambda i: (i, 0)),
        ],
        out_specs=pl.BlockSpec((BC, OT), lambda i: (0, i)),
        out_shape=jax.ShapeDtypeStruct((BC, O), jnp.float32),
    )(x2, M)
    return y.reshape(B, C, O)


# speedup vs baseline: 1.0332x; 1.0332x over previous
import jax
import jax.numpy as jnp
from jax.experimental import pallas as pl

_OT = 128


def _matmul_block(x_ref, m0_ref, m1_ref, o0_ref, o1_ref):
    x = x_ref[...]
    dn = (((1,), (1,)), ((), ()))
    o0_ref[...] = jax.lax.dot_general(
        x, m0_ref[...], dimension_numbers=dn,
        preferred_element_type=jnp.float32)
    o1_ref[...] = jax.lax.dot_general(
        x, m1_ref[...], dimension_numbers=dn,
        preferred_element_type=jnp.float32)


def kernel(x, M):
    B, C, N = x.shape
    O = M.shape[0]
    BC = B * C
    x2 = x.reshape(BC, N)

    steps = pl.cdiv(O, 2 * _OT)
    split = steps * _OT

    y0, y1 = pl.pallas_call(
        _matmul_block,
        grid=(steps,),
        in_specs=[
            pl.BlockSpec((BC, N), lambda i: (0, 0)),
            pl.BlockSpec((_OT, N), lambda i: (i, 0)),
            pl.BlockSpec((_OT, N), lambda i: (i + steps, 0)),
        ],
        out_specs=[
            pl.BlockSpec((BC, _OT), lambda i: (0, i)),
            pl.BlockSpec((BC, _OT), lambda i: (0, i)),
        ],
        out_shape=[
            jax.ShapeDtypeStruct((BC, split), jnp.float32),
            jax.ShapeDtypeStruct((BC, O - split), jnp.float32),
        ],
    )(x2, M, M)
    y = jnp.concatenate([y0, y1], axis=1)
    return y.reshape(B, C, O)
